# baseline (speedup 1.0000x reference)
import functools

import jax
import jax.numpy as jnp
from jax import lax
from jax.experimental import pallas as pl
from jax.experimental.pallas import tpu as pltpu
from jax.experimental.pallas import tpu_sc as plsc

_EPS = 1e-5
_CH = 128
_NCORES = 2
_NSUB = 16
_NW = _NCORES * _NSUB


def _softplus(z):
    return jnp.maximum(z, 0.0) + jnp.log1p(jnp.exp(-jnp.abs(z)))


def _sigmoid(z):
    return 1.0 / (1.0 + jnp.exp(-z))


@functools.lru_cache(maxsize=None)
def _make_gather(n_tab, d, n_chunks, dtype):
    cpw = n_chunks // _NW
    mesh = plsc.VectorSubcoreMesh(core_axis_name="c", subcore_axis_name="s")

    @functools.partial(
        pl.kernel,
        mesh=mesh,
        out_type=jax.ShapeDtypeStruct((n_chunks, _CH, d), dtype),
        scratch_types=[
            pltpu.VMEM((_CH,), jnp.int32),
            pltpu.VMEM((_CH, d), dtype),
            pltpu.SemaphoreType.DMA,
        ],
        compiler_params=pltpu.CompilerParams(use_tc_tiling_on_sc=False),
    )
    def gather_k(tab_hbm, idx_hbm, out_hbm, idx_v, rows_v, sem):
        wid = lax.axis_index("s") * _NCORES + lax.axis_index("c")
        c0 = wid * cpw

        @pl.loop(0, cpw)
        def _(j):
            g = c0 + j
            pltpu.sync_copy(idx_hbm.at[g], idx_v)
            pltpu.async_copy(tab_hbm.at[idx_v], rows_v, sem).wait()
            pltpu.sync_copy(rows_v, out_hbm.at[g])

    return gather_k


def _gather_rows(x, idx2):
    n_tab, d = x.shape
    n_chunks = idx2.shape[0]
    return _make_gather(n_tab, d, n_chunks, x.dtype)(x, idx2)


def _embed_body(af_ref, ew_ref, eb_ref, sw_ref, sb_ref, wn_ref,
                x_ref, sk_ref, xwn_ref, cm_ref, sc_ref):
    x = jnp.dot(af_ref[...], ew_ref[...],
                preferred_element_type=jnp.float32) + eb_ref[...]
    x_ref[...] = x
    sk_ref[...] = jnp.dot(x, sw_ref[...],
                          preferred_element_type=jnp.float32) + sb_ref[...]
    xwn = jnp.dot(x, wn_ref[...], preferred_element_type=jnp.float32)
    cm = jnp.mean(xwn, axis=0, keepdims=True)
    cm_ref[...] = cm
    xc = xwn - cm
    sc = jnp.max(jnp.abs(xc), axis=0, keepdims=True) / 32000.0 + 1e-20
    sc_ref[...] = sc
    xwn_ref[...] = jnp.round(xc / sc).astype(jnp.int16)


def _embed(atom_fea, emb_w, emb_b, skip_w, skip_b, wn0):
    n = atom_fea.shape[0]
    a = emb_w.shape[1]
    h2 = wn0.shape[1]
    return pl.pallas_call(
        _embed_body,
        out_shape=[jax.ShapeDtypeStruct((n, a), jnp.float32),
                   jax.ShapeDtypeStruct((n, a), jnp.float32),
                   jax.ShapeDtypeStruct((n, h2), jnp.int16),
                   jax.ShapeDtypeStruct((1, h2), jnp.float32),
                   jax.ShapeDtypeStruct((1, h2), jnp.float32)],
    )(atom_fea, emb_w, emb_b.reshape(1, a), skip_w, skip_b.reshape(1, a), wn0)


def _conv_stats_body(nblk_m, h2, x_ref, pk_ref, f_ref, ws_ref,
                     wf_ref, b_ref, cm_ref, sc_ref, s1_ref, s2_ref):
    i = pl.program_id(0)
    ba, m, nbr = f_ref.shape
    xg = jnp.dot(x_ref[...], ws_ref[...], preferred_element_type=jnp.float32)
    ng = pk_ref[...].reshape(ba * m, h2).astype(jnp.float32) * sc_ref[...]
    fg = jnp.dot(f_ref[...].reshape(ba * m, nbr), wf_ref[...],
                 preferred_element_type=jnp.float32)
    g = ((ng + fg).reshape(ba, m, h2) + xg[:, None, :]
         + (b_ref[...] + cm_ref[...])[None])

    @pl.when(i == 0)
    def _():
        s1_ref[...] = jnp.zeros_like(s1_ref)
        s2_ref[...] = jnp.zeros_like(s2_ref)

    s1_ref[...] += jnp.sum(g, axis=(0, 1))[None]
    s2_ref[...] += jnp.sum(g * g, axis=(0, 1))[None]


def _conv_apply_body(ne, h2, x_ref, pk_ref, f_ref, ws_ref, wf_ref,
                     b_ref, cm_ref, sc_ref, s1_ref, s2_ref, g1_ref, b1_ref,
                     sum_ref, t1_ref, t2_ref):
    i = pl.program_id(0)
    ba, m, nbr = f_ref.shape
    a = x_ref.shape[1]
    xg = jnp.dot(x_ref[...], ws_ref[...], preferred_element_type=jnp.float32)
    ng = pk_ref[...].reshape(ba * m, h2).astype(jnp.float32) * sc_ref[...]
    fg = jnp.dot(f_ref[...].reshape(ba * m, nbr), wf_ref[...],
                 preferred_element_type=jnp.float32)
    g = ((ng + fg).reshape(ba, m, h2) + xg[:, None, :]
         + (b_ref[...] + cm_ref[...])[None])

    mu = s1_ref[...] / ne
    var = s2_ref[...] / ne - mu * mu
    scale = g1_ref[...] * lax.rsqrt(var + _EPS)
    shift = b1_ref[...] - mu * scale
    gn = g * scale[None] + shift[None]

    filt = _sigmoid(gn[:, :, :a])
    core = _softplus(gn[:, :, a:])
    summed = jnp.sum(filt * core, axis=1)
    sum_ref[...] = summed

    @pl.when(i == 0)
    def _():
        t1_ref[...] = jnp.zeros_like(t1_ref)
        t2_ref[...] = jnp.zeros_like(t2_ref)

    t1_ref[...] += jnp.sum(summed, axis=0)[None]
    t2_ref[...] += jnp.sum(summed * summed, axis=0)[None]


def _conv_finish_body(n, x_ref, s_ref, t1_ref, t2_ref, g2_ref, b2_ref,
                      o_ref):
    mu = t1_ref[...] / n
    var = t2_ref[...] / n - mu * mu
    scale = g2_ref[...] * lax.rsqrt(var + _EPS)
    shift = b2_ref[...] - mu * scale
    x = x_ref[...]
    o_ref[...] = _softplus(x + s_ref[...] * scale + shift) + x


def _conv_finish_next_body(n, x_ref, s_ref, t1_ref, t2_ref, g2_ref, b2_ref,
                           wn_ref, o_ref, xwn_ref, cm_ref, sc_ref):
    mu = t1_ref[...] / n
    var = t2_ref[...] / n - mu * mu
    scale = g2_ref[...] * lax.rsqrt(var + _EPS)
    shift = b2_ref[...] - mu * scale
    x = x_ref[...]
    o = _softplus(x + s_ref[...] * scale + shift) + x
    o_ref[...] = o
    xwn = jnp.dot(o, wn_ref[...], preferred_element_type=jnp.float32)
    cm = jnp.mean(xwn, axis=0, keepdims=True)
    cm_ref[...] = cm
    xc = xwn - cm
    sc = jnp.max(jnp.abs(xc), axis=0, keepdims=True) / 32000.0 + 1e-20
    sc_ref[...] = sc
    xwn_ref[...] = jnp.round(xc / sc).astype(jnp.int16)


def _conv_layer(x, packed, cm, scq, nbr_b, cp, ba, wn_next):
    n, a = x.shape
    _, m, nbr = nbr_b.shape
    h2 = 2 * a
    nblk = n // ba
    cpb = (ba * m) // _CH
    ne = float(n * m)

    ws = cp["fc_W"][:a]
    wf = cp["fc_W"][2 * a:]
    b = cp["fc_b"].reshape(1, h2)
    g1 = cp["bn1_g"].reshape(1, h2)
    b1 = cp["bn1_b"].reshape(1, h2)
    g2 = cp["bn2_g"].reshape(1, a)
    b2 = cp["bn2_b"].reshape(1, a)

    common_specs = [
        pl.BlockSpec((ba, a), lambda i: (i, 0)),
        pl.BlockSpec((cpb, _CH, h2), lambda i: (i, 0, 0)),
        pl.BlockSpec((ba, m, nbr), lambda i: (i, 0, 0)),
        pl.BlockSpec((a, h2), lambda i: (0, 0)),
        pl.BlockSpec((nbr, h2), lambda i: (0, 0)),
        pl.BlockSpec((1, h2), lambda i: (0, 0)),
        pl.BlockSpec((1, h2), lambda i: (0, 0)),
        pl.BlockSpec((1, h2), lambda i: (0, 0)),
    ]
    vec_spec = pl.BlockSpec((1, h2), lambda i: (0, 0))
    vec_a_spec = pl.BlockSpec((1, a), lambda i: (0, 0))

    s1, s2 = pl.pallas_call(
        functools.partial(_conv_stats_body, nblk, h2),
        grid=(nblk,),
        in_specs=common_specs,
        out_specs=[vec_spec, vec_spec],
        out_shape=[jax.ShapeDtypeStruct((1, h2), jnp.float32)] * 2,
        compiler_params=pltpu.CompilerParams(
            dimension_semantics=("arbitrary",)),
    )(x, packed, nbr_b, ws, wf, b, cm, scq)

    summed, t1, t2 = pl.pallas_call(
        functools.partial(_conv_apply_body, ne, h2),
        grid=(nblk,),
        in_specs=common_specs + [vec_spec] * 4,
        out_specs=[pl.BlockSpec((ba, a), lambda i: (i, 0)),
                   vec_a_spec, vec_a_spec],
        out_shape=[jax.ShapeDtypeStruct((n, a), jnp.float32),
                   jax.ShapeDtypeStruct((1, a), jnp.float32),
                   jax.ShapeDtypeStruct((1, a), jnp.float32)],
        compiler_params=pltpu.CompilerParams(
            dimension_semantics=("arbitrary",)),
    )(x, packed, nbr_b, ws, wf, b, cm, scq, s1, s2, g1, b1)

    if wn_next is None:
        xn = pl.pallas_call(
            functools.partial(_conv_finish_body, float(n)),
            out_shape=jax.ShapeDtypeStruct((n, a), jnp.float32),
        )(x, summed, t1, t2, g2, b2)
        return xn, None, None, None
    xn, xwn, cm_n, sc_n = pl.pallas_call(
        functools.partial(_conv_finish_next_body, float(n)),
        out_shape=[jax.ShapeDtypeStruct((n, a), jnp.float32),
                   jax.ShapeDtypeStruct((n, h2), jnp.int16),
                   jax.ShapeDtypeStruct((1, h2), jnp.float32),
                   jax.ShapeDtypeStruct((1, h2), jnp.float32)],
    )(x, summed, t1, t2, g2, b2, wn_next)
    return xn, xwn, cm_n, sc_n


def _pool_body(n0, per, cw_ref, cb_ref, ow_ref, ob_ref, x_ref, sk_ref, o_ref):
    n = x_ref.shape[0]
    xf = x_ref[...] + sk_ref[...]
    rows = lax.broadcasted_iota(jnp.int32, (n0, n), 0)
    cols = lax.broadcasted_iota(jnp.int32, (n0, n), 1)
    msk = jnp.where(cols // per == rows, 1.0 / per, 0.0)
    pooled = jnp.dot(msk, xf, preferred_element_type=jnp.float32)
    h = _softplus(pooled)
    h = jnp.dot(h, cw_ref[...], preferred_element_type=jnp.float32) + cb_ref[...]
    h = _softplus(h)
    o_ref[...] = jnp.dot(h, ow_ref[...],
                         preferred_element_type=jnp.float32) + ob_ref[...]


def _pool(x, skip, n0, per, c2f_w, c2f_b, out_w, out_b):
    h = c2f_w.shape[1]
    return pl.pallas_call(
        functools.partial(_pool_body, n0, per),
        out_shape=jax.ShapeDtypeStruct((n0, out_w.shape[1]), jnp.float32),
    )(c2f_w, c2f_b.reshape(1, h), out_w, out_b.reshape(1, out_w.shape[1]),
      x, skip)


def kernel(atom_fea, nbr_fea, nbr_fea_idx, crystal_atom_idx, params):
    n, _ = atom_fea.shape
    _, m, _ = nbr_fea.shape
    a = params["emb_W"].shape[1]
    n0, per = crystal_atom_idx.shape

    ba = 400
    e = n * m
    n_chunks = -(-e // _CH)
    n_chunks = -(-n_chunks // _NW) * _NW

    idx_flat = nbr_fea_idx.reshape(-1).astype(jnp.int32)
    pad = n_chunks * _CH - e
    idx2 = jnp.concatenate(
        [idx_flat, jnp.zeros((pad,), jnp.int32)]).reshape(n_chunks, _CH)

    wns = [cp["fc_W"][a:2 * a] for cp in params["convs"]]

    x, skip, xwn, cm, scq = _embed(atom_fea, params["emb_W"],
                                   params["emb_b"], params["skip_W"],
                                   params["skip_b"], wns[0])

    for li, cp in enumerate(params["convs"]):
        packed = _gather_rows(xwn, idx2)
        wn_next = wns[li + 1] if li + 1 < len(params["convs"]) else None
        x, xwn, cm, scq = _conv_layer(x, packed, cm, scq, nbr_fea, cp, ba,
                                      wn_next)

    return _pool(x, skip, n0, per, params["c2f_W"], params["c2f_b"],
                 params["out_W"], params["out_b"])


# speedup vs baseline: 1.1266x; 1.1266x over previous
import functools

import jax
import jax.numpy as jnp
from jax import lax
from jax.experimental import pallas as pl
from jax.experimental.pallas import tpu as pltpu
from jax.experimental.pallas import tpu_sc as plsc

_EPS = 1e-5
_CH = 128
_NCORES = 2
_NSUB = 16
_NW = _NCORES * _NSUB


def _softplus(z):
    return jnp.maximum(z, 0.0) + jnp.log1p(jnp.exp(-jnp.abs(z)))


def _sigmoid(z):
    return 1.0 / (1.0 + jnp.exp(-z))


@functools.lru_cache(maxsize=None)
def _make_gather(n_tab, d, n_chunks):
    cpw = n_chunks // _NW
    mesh = plsc.VectorSubcoreMesh(core_axis_name="c", subcore_axis_name="s")

    @functools.partial(
        pl.kernel,
        mesh=mesh,
        out_type=jax.ShapeDtypeStruct((n_chunks, _CH, d), jnp.float32),
        scratch_types=[
            pltpu.VMEM((_CH,), jnp.int32),
            pltpu.VMEM((_CH, d), jnp.float32),
            pltpu.SemaphoreType.DMA,
        ],
        compiler_params=pltpu.CompilerParams(use_tc_tiling_on_sc=False),
    )
    def gather_k(tab_hbm, idx_hbm, out_hbm, idx_v, rows_v, sem):
        wid = lax.axis_index("s") * _NCORES + lax.axis_index("c")
        c0 = wid * cpw

        @pl.loop(0, cpw)
        def _(j):
            g = c0 + j
            pltpu.sync_copy(idx_hbm.at[g], idx_v)
            pltpu.async_copy(tab_hbm.at[idx_v], rows_v, sem).wait()
            pltpu.sync_copy(rows_v, out_hbm.at[g])

    return gather_k


def _gather_rows(x, idx2):
    n_tab, d = x.shape
    n_chunks = idx2.shape[0]
    return _make_gather(n_tab, d, n_chunks)(x, idx2)


def _embed_body(af_ref, ew_ref, eb_ref, sw_ref, sb_ref, x_ref, sk_ref):
    x = jnp.dot(af_ref[...], ew_ref[...],
                preferred_element_type=jnp.float32) + eb_ref[...]
    x_ref[...] = x
    sk_ref[...] = jnp.dot(x, sw_ref[...],
                          preferred_element_type=jnp.float32) + sb_ref[...]


def _embed(atom_fea, emb_w, emb_b, skip_w, skip_b):
    n = atom_fea.shape[0]
    a = emb_w.shape[1]
    return pl.pallas_call(
        _embed_body,
        out_shape=[jax.ShapeDtypeStruct((n, a), jnp.float32),
                   jax.ShapeDtypeStruct((n, a), jnp.float32)],
    )(atom_fea, emb_w, emb_b.reshape(1, a), skip_w, skip_b.reshape(1, a))


def _conv_stats_body(nblk_m, h2, x_ref, nb_ref, f_ref, ws_ref, wn_ref,
                     wf_ref, b_ref, s1_ref, s2_ref):
    i = pl.program_id(0)
    ba, m, nbr = f_ref.shape
    xg = jnp.dot(x_ref[...], ws_ref[...], preferred_element_type=jnp.float32)
    nb = nb_ref[...].reshape(ba * m, x_ref.shape[1])
    ng = jnp.dot(nb, wn_ref[...], preferred_element_type=jnp.float32)
    fg = jnp.dot(f_ref[...].reshape(ba * m, nbr), wf_ref[...],
                 preferred_element_type=jnp.float32)
    g = (ng + fg).reshape(ba, m, h2) + xg[:, None, :] + b_ref[...][None]

    @pl.when(i == 0)
    def _():
        s1_ref[...] = jnp.zeros_like(s1_ref)
        s2_ref[...] = jnp.zeros_like(s2_ref)

    s1_ref[...] += jnp.sum(g, axis=(0, 1))[None]
    s2_ref[...] += jnp.sum(g * g, axis=(0, 1))[None]


def _conv_apply_body(ne, h2, x_ref, nb_ref, f_ref, ws_ref, wn_ref, wf_ref,
                     b_ref, s1a_ref, s2a_ref, s1b_ref, s2b_ref,
                     g1_ref, b1_ref, sum_ref, t1_ref, t2_ref):
    i = pl.program_id(0)
    ba, m, nbr = f_ref.shape
    a = x_ref.shape[1]
    xg = jnp.dot(x_ref[...], ws_ref[...], preferred_element_type=jnp.float32)
    nb = nb_ref[...].reshape(ba * m, a)
    ng = jnp.dot(nb, wn_ref[...], preferred_element_type=jnp.float32)
    fg = jnp.dot(f_ref[...].reshape(ba * m, nbr), wf_ref[...],
                 preferred_element_type=jnp.float32)
    g = (ng + fg).reshape(ba, m, h2) + xg[:, None, :] + b_ref[...][None]

    mu = (s1a_ref[...] + s1b_ref[...]) / ne
    var = (s2a_ref[...] + s2b_ref[...]) / ne - mu * mu
    scale = g1_ref[...] * lax.rsqrt(var + _EPS)
    shift = b1_ref[...] - mu * scale
    gn = g * scale[None] + shift[None]

    filt = _sigmoid(gn[:, :, :a])
    core = _softplus(gn[:, :, a:])
    summed = jnp.sum(filt * core, axis=1)
    sum_ref[...] = summed

    @pl.when(i == 0)
    def _():
        t1_ref[...] = jnp.zeros_like(t1_ref)
        t2_ref[...] = jnp.zeros_like(t2_ref)

    t1_ref[...] += jnp.sum(summed, axis=0)[None]
    t2_ref[...] += jnp.sum(summed * summed, axis=0)[None]


def _conv_finish_body(n, x_ref, sa_ref, sb_ref, t1a_ref, t2a_ref,
                      t1b_ref, t2b_ref, g2_ref, b2_ref, o_ref):
    t1 = t1a_ref[...] + t1b_ref[...]
    t2 = t2a_ref[...] + t2b_ref[...]
    mu = t1 / n
    var = t2 / n - mu * mu
    scale = g2_ref[...] * lax.rsqrt(var + _EPS)
    shift = b2_ref[...] - mu * scale
    x = x_ref[...]
    s = jnp.concatenate([sa_ref[...], sb_ref[...]], axis=0)
    o_ref[...] = _softplus(x + s * scale + shift) + x


def _conv_layer(x, pk_a, pk_b, nbr_fea, cp, ba):
    n, a = x.shape
    _, m, nbr = nbr_fea.shape
    h2 = 2 * a
    half = n // 2
    nblkh = half // ba
    cpb = (ba * m) // _CH
    ne = float(n * m)

    ws = cp["fc_W"][:a]
    wn = cp["fc_W"][a:2 * a]
    wf = cp["fc_W"][2 * a:]
    b = cp["fc_b"].reshape(1, h2)
    g1 = cp["bn1_g"].reshape(1, h2)
    b1 = cp["bn1_b"].reshape(1, h2)
    g2 = cp["bn2_g"].reshape(1, a)
    b2 = cp["bn2_b"].reshape(1, a)

    def specs(off):
        return [
            pl.BlockSpec((ba, a), lambda i: (i + off, 0)),
            pl.BlockSpec((cpb, _CH, a), lambda i: (i, 0, 0)),
            pl.BlockSpec((ba, m, nbr), lambda i: (i + off, 0, 0)),
            pl.BlockSpec((a, h2), lambda i: (0, 0)),
            pl.BlockSpec((a, h2), lambda i: (0, 0)),
            pl.BlockSpec((nbr, h2), lambda i: (0, 0)),
            pl.BlockSpec((1, h2), lambda i: (0, 0)),
        ]

    vec_spec = pl.BlockSpec((1, h2), lambda i: (0, 0))
    vec_a_spec = pl.BlockSpec((1, a), lambda i: (0, 0))
    stats_kw = dict(
        grid=(nblkh,),
        out_specs=[vec_spec, vec_spec],
        out_shape=[jax.ShapeDtypeStruct((1, h2), jnp.float32)] * 2,
        compiler_params=pltpu.CompilerParams(
            dimension_semantics=("arbitrary",)),
    )
    s1a, s2a = pl.pallas_call(
        functools.partial(_conv_stats_body, nblkh, h2),
        in_specs=specs(0), **stats_kw,
    )(x, pk_a, nbr_fea, ws, wn, wf, b)
    s1b, s2b = pl.pallas_call(
        functools.partial(_conv_stats_body, nblkh, h2),
        in_specs=specs(nblkh), **stats_kw,
    )(x, pk_b, nbr_fea, ws, wn, wf, b)

    apply_kw = dict(
        grid=(nblkh,),
        out_specs=[pl.BlockSpec((ba, a), lambda i: (i, 0)),
                   vec_a_spec, vec_a_spec],
        out_shape=[jax.ShapeDtypeStruct((half, a), jnp.float32),
                   jax.ShapeDtypeStruct((1, a), jnp.float32),
                   jax.ShapeDtypeStruct((1, a), jnp.float32)],
        compiler_params=pltpu.CompilerParams(
            dimension_semantics=("arbitrary",)),
    )
    sum_a, t1a, t2a = pl.pallas_call(
        functools.partial(_conv_apply_body, ne, h2),
        in_specs=specs(0) + [vec_spec] * 4 + [vec_spec] * 2,
        **apply_kw,
    )(x, pk_a, nbr_fea, ws, wn, wf, b, s1a, s2a, s1b, s2b, g1, b1)
    sum_b, t1b, t2b = pl.pallas_call(
        functools.partial(_conv_apply_body, ne, h2),
        in_specs=specs(nblkh) + [vec_spec] * 4 + [vec_spec] * 2,
        **apply_kw,
    )(x, pk_b, nbr_fea, ws, wn, wf, b, s1a, s2a, s1b, s2b, g1, b1)

    return pl.pallas_call(
        functools.partial(_conv_finish_body, float(n)),
        out_shape=jax.ShapeDtypeStruct((n, a), jnp.float32),
    )(x, sum_a, sum_b, t1a, t2a, t1b, t2b, g2, b2)


def _pool_body(n0, per, cw_ref, cb_ref, ow_ref, ob_ref, x_ref, sk_ref, o_ref):
    n = x_ref.shape[0]
    xf = x_ref[...] + sk_ref[...]
    rows = lax.broadcasted_iota(jnp.int32, (n0, n), 0)
    cols = lax.broadcasted_iota(jnp.int32, (n0, n), 1)
    msk = jnp.where(cols // per == rows, 1.0 / per, 0.0)
    pooled = jnp.dot(msk, xf, preferred_element_type=jnp.float32)
    h = _softplus(pooled)
    h = jnp.dot(h, cw_ref[...], preferred_element_type=jnp.float32) + cb_ref[...]
    h = _softplus(h)
    o_ref[...] = jnp.dot(h, ow_ref[...],
                         preferred_element_type=jnp.float32) + ob_ref[...]


def _pool(x, skip, n0, per, c2f_w, c2f_b, out_w, out_b):
    h = c2f_w.shape[1]
    return pl.pallas_call(
        functools.partial(_pool_body, n0, per),
        out_shape=jax.ShapeDtypeStruct((n0, out_w.shape[1]), jnp.float32),
    )(c2f_w, c2f_b.reshape(1, h), out_w, out_b.reshape(1, out_w.shape[1]),
      x, skip)


def kernel(atom_fea, nbr_fea, nbr_fea_idx, crystal_atom_idx, params):
    n, _ = atom_fea.shape
    _, m, _ = nbr_fea.shape
    a = params["emb_W"].shape[1]
    n0, per = crystal_atom_idx.shape

    ba = 200
    half_e = (n // 2) * m
    nch = half_e // _CH
    nch_pad = -(-nch // _NW) * _NW

    idx_flat = nbr_fea_idx.reshape(-1).astype(jnp.int32)
    pad = jnp.zeros((nch_pad * _CH - half_e,), jnp.int32)
    idx_a = jnp.concatenate(
        [idx_flat[:half_e], pad]).reshape(nch_pad, _CH)
    idx_b = jnp.concatenate(
        [idx_flat[half_e:], pad]).reshape(nch_pad, _CH)

    x, skip = _embed(atom_fea, params["emb_W"], params["emb_b"],
                     params["skip_W"], params["skip_b"])

    for cp in params["convs"]:
        pk_a = _gather_rows(x, idx_a)
        pk_b = _gather_rows(x, idx_b)
        x = _conv_layer(x, pk_a, pk_b, nbr_fea, cp, ba)

    return _pool(x, skip, n0, per, params["c2f_W"], params["c2f_b"],
                 params["out_W"], params["out_b"])


# speedup vs baseline: 1.2891x; 1.1443x over previous
import functools

import jax
import jax.numpy as jnp
from jax import lax
from jax.experimental import pallas as pl
from jax.experimental.pallas import tpu as pltpu
from jax.experimental.pallas import tpu_sc as plsc

_EPS = 1e-5
_CH = 128
_NCORES = 2
_NSUB = 16
_NW = _NCORES * _NSUB


def _softplus(z):
    return jnp.maximum(z, 0.0) + jnp.log1p(jnp.exp(-jnp.abs(z)))


def _sigmoid(z):
    return 1.0 / (1.0 + jnp.exp(-z))


@functools.lru_cache(maxsize=None)
def _make_gather(n_tab, d, n_chunks):
    cpw = n_chunks // _NW
    mesh = plsc.VectorSubcoreMesh(core_axis_name="c", subcore_axis_name="s")

    @functools.partial(
        pl.kernel,
        mesh=mesh,
        out_type=jax.ShapeDtypeStruct((n_chunks, _CH, d), jnp.float32),
        scratch_types=[
            pltpu.VMEM((_CH,), jnp.int32),
            pltpu.VMEM((_CH, d), jnp.float32),
            pltpu.SemaphoreType.DMA,
        ],
        compiler_params=pltpu.CompilerParams(use_tc_tiling_on_sc=False),
    )
    def gather_k(tab_hbm, idx_hbm, out_hbm, idx_v, rows_v, sem):
        wid = lax.axis_index("s") * _NCORES + lax.axis_index("c")
        c0 = wid * cpw

        @pl.loop(0, cpw)
        def _(j):
            g = c0 + j
            pltpu.sync_copy(idx_hbm.at[g], idx_v)
            pltpu.async_copy(tab_hbm.at[idx_v], rows_v, sem).wait()
            pltpu.sync_copy(rows_v, out_hbm.at[g])

    return gather_k


def _gather_rows(x, idx2):
    n_tab, d = x.shape
    n_chunks = idx2.shape[0]
    return _make_gather(n_tab, d, n_chunks)(x, idx2)


def _embed_body(af_ref, ew_ref, eb_ref, sw_ref, sb_ref, x_ref, sk_ref):
    x = jnp.dot(af_ref[...], ew_ref[...],
                preferred_element_type=jnp.float32) + eb_ref[...]
    x_ref[...] = x
    sk_ref[...] = jnp.dot(x, sw_ref[...],
                          preferred_element_type=jnp.float32) + sb_ref[...]


def _embed(atom_fea, emb_w, emb_b, skip_w, skip_b):
    n = atom_fea.shape[0]
    a = emb_w.shape[1]
    return pl.pallas_call(
        _embed_body,
        out_shape=[jax.ShapeDtypeStruct((n, a), jnp.float32),
                   jax.ShapeDtypeStruct((n, a), jnp.float32)],
    )(atom_fea, emb_w, emb_b.reshape(1, a), skip_w, skip_b.reshape(1, a))


def _conv_stats_body(nblk_m, h2, x_ref, nb_ref, f_ref, ws_ref, wn_ref,
                     wf_ref, b_ref, s1_ref, s2_ref):
    i = pl.program_id(0)
    ba, m, nbr = f_ref.shape
    xg = jnp.dot(x_ref[...], ws_ref[...], preferred_element_type=jnp.float32)
    nb = nb_ref[...].reshape(ba * m, x_ref.shape[1])
    ng = jnp.dot(nb, wn_ref[...], preferred_element_type=jnp.float32)
    fg = jnp.dot(f_ref[...].reshape(ba * m, nbr), wf_ref[...],
                 preferred_element_type=jnp.float32)
    g = (ng + fg).reshape(ba, m, h2) + xg[:, None, :] + b_ref[...][None]

    @pl.when(i == 0)
    def _():
        s1_ref[...] = jnp.zeros_like(s1_ref)
        s2_ref[...] = jnp.zeros_like(s2_ref)

    s1_ref[...] += jnp.sum(g, axis=(0, 1))[None]
    s2_ref[...] += jnp.sum(g * g, axis=(0, 1))[None]


def _conv_apply_body(ne, h2, x_ref, nb_ref, f_ref, ws_ref, wn_ref, wf_ref,
                     b_ref, s1_ref, s2_ref, g1_ref, b1_ref,
                     sum_ref, t1_ref, t2_ref):
    i = pl.program_id(0)
    ba, m, nbr = f_ref.shape
    a = x_ref.shape[1]
    xg = jnp.dot(x_ref[...], ws_ref[...], preferred_element_type=jnp.float32)
    nb = nb_ref[...].reshape(ba * m, a)
    ng = jnp.dot(nb, wn_ref[...], preferred_element_type=jnp.float32)
    fg = jnp.dot(f_ref[...].reshape(ba * m, nbr), wf_ref[...],
                 preferred_element_type=jnp.float32)
    g = (ng + fg).reshape(ba, m, h2) + xg[:, None, :] + b_ref[...][None]

    mu = s1_ref[...] / ne
    var = s2_ref[...] / ne - mu * mu
    scale = g1_ref[...] * lax.rsqrt(var + _EPS)
    shift = b1_ref[...] - mu * scale
    gn = g * scale[None] + shift[None]

    filt = _sigmoid(gn[:, :, :a])
    core = _softplus(gn[:, :, a:])
    summed = jnp.sum(filt * core, axis=1)
    sum_ref[...] = summed

    @pl.when(i == 0)
    def _():
        t1_ref[...] = jnp.zeros_like(t1_ref)
        t2_ref[...] = jnp.zeros_like(t2_ref)

    t1_ref[...] += jnp.sum(summed, axis=0)[None]
    t2_ref[...] += jnp.sum(summed * summed, axis=0)[None]


def _conv_finish_body(n, x_ref, s_ref, t1_ref, t2_ref, g2_ref, b2_ref, o_ref):
    mu = t1_ref[...] / n
    var = t2_ref[...] / n - mu * mu
    scale = g2_ref[...] * lax.rsqrt(var + _EPS)
    shift = b2_ref[...] - mu * scale
    x = x_ref[...]
    o_ref[...] = _softplus(x + s_ref[...] * scale + shift) + x


def _conv_layer(x, packed, nbr_fea, cp, ba):
    n, a = x.shape
    _, m, nbr = nbr_fea.shape
    h2 = 2 * a
    nblk = n // ba
    cpb = (ba * m) // _CH
    ne = float(n * m)

    ws = cp["fc_W"][:a]
    wn = cp["fc_W"][a:2 * a]
    wf = cp["fc_W"][2 * a:]
    b = cp["fc_b"].reshape(1, h2)
    g1 = cp["bn1_g"].reshape(1, h2)
    b1 = cp["bn1_b"].reshape(1, h2)
    g2 = cp["bn2_g"].reshape(1, a)
    b2 = cp["bn2_b"].reshape(1, a)

    common_specs = [
        pl.BlockSpec((ba, a), lambda i: (i, 0)),
        pl.BlockSpec((cpb, _CH, a), lambda i: (i, 0, 0)),
        pl.BlockSpec((ba, m, nbr), lambda i: (i, 0, 0)),
        pl.BlockSpec((a, h2), lambda i: (0, 0)),
        pl.BlockSpec((a, h2), lambda i: (0, 0)),
        pl.BlockSpec((nbr, h2), lambda i: (0, 0)),
        pl.BlockSpec((1, h2), lambda i: (0, 0)),
    ]
    vec_spec = pl.BlockSpec((1, h2), lambda i: (0, 0))
    vec_a_spec = pl.BlockSpec((1, a), lambda i: (0, 0))

    s1, s2 = pl.pallas_call(
        functools.partial(_conv_stats_body, nblk, h2),
        grid=(nblk,),
        in_specs=common_specs,
        out_specs=[vec_spec, vec_spec],
        out_shape=[jax.ShapeDtypeStruct((1, h2), jnp.float32)] * 2,
        compiler_params=pltpu.CompilerParams(
            dimension_semantics=("arbitrary",)),
    )(x, packed, nbr_fea, ws, wn, wf, b)

    summed, t1, t2 = pl.pallas_call(
        functools.partial(_conv_apply_body, ne, h2),
        grid=(nblk,),
        in_specs=common_specs + [vec_spec] * 4,
        out_specs=[pl.BlockSpec((ba, a), lambda i: (i, 0)),
                   vec_a_spec, vec_a_spec],
        out_shape=[jax.ShapeDtypeStruct((n, a), jnp.float32),
                   jax.ShapeDtypeStruct((1, a), jnp.float32),
                   jax.ShapeDtypeStruct((1, a), jnp.float32)],
        compiler_params=pltpu.CompilerParams(
            dimension_semantics=("arbitrary",)),
    )(x, packed, nbr_fea, ws, wn, wf, b, s1, s2, g1, b1)

    return pl.pallas_call(
        functools.partial(_conv_finish_body, float(n)),
        out_shape=jax.ShapeDtypeStruct((n, a), jnp.float32),
    )(x, summed, t1, t2, g2, b2)


def _pool_body(n0, per, cw_ref, cb_ref, ow_ref, ob_ref, x_ref, sk_ref, o_ref):
    n = x_ref.shape[0]
    xf = x_ref[...] + sk_ref[...]
    rows = lax.broadcasted_iota(jnp.int32, (n0, n), 0)
    cols = lax.broadcasted_iota(jnp.int32, (n0, n), 1)
    msk = jnp.where(cols // per == rows, 1.0 / per, 0.0)
    pooled = jnp.dot(msk, xf, preferred_element_type=jnp.float32)
    h = _softplus(pooled)
    h = jnp.dot(h, cw_ref[...], preferred_element_type=jnp.float32) + cb_ref[...]
    h = _softplus(h)
    o_ref[...] = jnp.dot(h, ow_ref[...],
                         preferred_element_type=jnp.float32) + ob_ref[...]


def _pool(x, skip, n0, per, c2f_w, c2f_b, out_w, out_b):
    h = c2f_w.shape[1]
    return pl.pallas_call(
        functools.partial(_pool_body, n0, per),
        out_shape=jax.ShapeDtypeStruct((n0, out_w.shape[1]), jnp.float32),
    )(c2f_w, c2f_b.reshape(1, h), out_w, out_b.reshape(1, out_w.shape[1]),
      x, skip)


def kernel(atom_fea, nbr_fea, nbr_fea_idx, crystal_atom_idx, params):
    n, _ = atom_fea.shape
    _, m, _ = nbr_fea.shape
    a = params["emb_W"].shape[1]
    n0, per = crystal_atom_idx.shape

    ba = 400
    e = n * m
    n_chunks = -(-e // _CH)
    n_chunks = -(-n_chunks // _NW) * _NW

    idx_flat = nbr_fea_idx.reshape(-1).astype(jnp.int32)
    pad = n_chunks * _CH - e
    idx2 = jnp.concatenate(
        [idx_flat, jnp.zeros((pad,), jnp.int32)]).reshape(n_chunks, _CH)

    x, skip = _embed(atom_fea, params["emb_W"], params["emb_b"],
                     params["skip_W"], params["skip_b"])

    for cp in params["convs"]:
        packed = _gather_rows(x, idx2)
        x = _conv_layer(x, packed, nbr_fea, cp, ba)

    return _pool(x, skip, n0, per, params["c2f_W"], params["c2f_b"],
                 params["out_W"], params["out_b"])
